# Initial kernel scaffold; baseline (speedup 1.0000x reference)
import jax
import jax.numpy as jnp
from jax.experimental import pallas as pl


def kernel(x, edge_index, W0, b0, Wm, bm, Wl, bl):
    raise NotImplementedError("write your pallas kernel here")


# speedup vs baseline: 16.8927x; 16.8927x over previous
import dataclasses
import functools

import jax
import jax.numpy as jnp
from jax import lax
from jax.experimental import pallas as pl
from jax.experimental.pallas import tpu as pltpu
from jax.experimental.pallas import tpu_sc as plsc

N_NODES = 10000
N_EDGES = 320000
F = 128
OUT_CH = 64

NC, NS = 2, 16
NW = NC * NS
EPW = N_EDGES // NW
CHUNK = 80
NCHUNK = EPW // CHUNK
N_PAD = 10240
RPS = N_PAD // NS
ZROWS = 128

_mesh = plsc.VectorSubcoreMesh(core_axis_name="c", subcore_axis_name="s")

_sc_params = pltpu.CompilerParams()
if "needs_layout_passes" in pltpu.CompilerParams.__dataclass_fields__:
    _sc_params = dataclasses.replace(_sc_params, needs_layout_passes=False)


@functools.partial(
    pl.kernel,
    out_type=[jax.ShapeDtypeStruct((N_PAD,), jnp.float32),
              jax.ShapeDtypeStruct((N_PAD,), jnp.float32)],
    mesh=_mesh,
    scratch_types=[
        pltpu.VMEM((EPW,), jnp.int32),
        pltpu.VMEM((N_PAD,), jnp.float32),
        pltpu.VMEM((RPS,), jnp.float32),
        pltpu.VMEM((RPS,), jnp.float32),
        pltpu.VMEM_SHARED((NS, N_PAD), jnp.float32),
    ],
    compiler_params=_sc_params,
)
def _deg_kernel(dst_hbm, out0_hbm, out1_hbm, dst_v, hist_v, red_v, tmp_v,
                acc_sh):
    c = lax.axis_index("c")
    s = lax.axis_index("s")
    wid = s * NC + c

    pltpu.sync_copy(dst_hbm.at[pl.ds(wid * EPW, EPW)], dst_v)

    @pl.loop(0, N_PAD // 16)
    def _(i):
        hist_v[pl.ds(i * 16, 16)] = jnp.zeros((16,), jnp.float32)

    ones16 = jnp.ones((16,), jnp.float32)

    @pl.loop(0, EPW // 16)
    def _(i):
        idx = dst_v[pl.ds(i * 16, 16)]
        plsc.addupdate_scatter(hist_v, [idx], ones16)

    pltpu.sync_copy(hist_v, acc_sh.at[s])
    plsc.subcore_barrier()

    @pl.loop(0, RPS // 16)
    def _(j):
        red_v[pl.ds(j * 16, 16)] = jnp.zeros((16,), jnp.float32)

    for k in range(NS):
        pltpu.sync_copy(acc_sh.at[k, pl.ds(s * RPS, RPS)], tmp_v)

        @pl.loop(0, RPS // 16)
        def _(j):
            red_v[pl.ds(j * 16, 16)] += tmp_v[pl.ds(j * 16, 16)]

    @pl.when(c == 0)
    def _():
        pltpu.sync_copy(red_v, out0_hbm.at[pl.ds(s * RPS, RPS)])

    @pl.when(c == 1)
    def _():
        pltpu.sync_copy(red_v, out1_hbm.at[pl.ds(s * RPS, RPS)])


@functools.partial(
    pl.kernel,
    out_type=[jax.ShapeDtypeStruct((N_PAD, F), jnp.float32),
              jax.ShapeDtypeStruct((N_PAD, F), jnp.float32)],
    mesh=_mesh,
    scratch_types=[
        pltpu.VMEM((CHUNK,), jnp.int32),
        pltpu.VMEM((CHUNK,), jnp.int32),
        pltpu.VMEM((CHUNK, F), jnp.float32),
        pltpu.VMEM((ZROWS, F), jnp.float32),
        pltpu.VMEM_SHARED((N_PAD, F), jnp.float32),
    ],
)
def _scat_kernel(g_hbm, src_hbm, dst_hbm, out0_hbm, out1_hbm, si_v, di_v,
                 rows_v, zero_v, acc_sh):
    c = lax.axis_index("c")
    s = lax.axis_index("s")
    wid = s * NC + c

    @pl.loop(0, ZROWS)
    def _(r):
        for j in range(F // 16):
            zero_v[r, pl.ds(j * 16, 16)] = jnp.zeros((16,), jnp.float32)

    @pl.loop(0, RPS // ZROWS)
    def _(b):
        pltpu.sync_copy(zero_v, acc_sh.at[pl.ds(s * RPS + b * ZROWS, ZROWS)])

    plsc.subcore_barrier()

    base = wid * EPW

    @pl.loop(0, NCHUNK)
    def _(ci):
        off = base + ci * CHUNK
        pltpu.sync_copy(src_hbm.at[pl.ds(off, CHUNK)], si_v)
        pltpu.sync_copy(dst_hbm.at[pl.ds(off, CHUNK)], di_v)
        pltpu.sync_copy(g_hbm.at[si_v], rows_v)
        pltpu.sync_copy(rows_v, acc_sh.at[di_v], add=True)

    plsc.subcore_barrier()

    @pl.when(c == 0)
    def _():
        pltpu.sync_copy(acc_sh.at[pl.ds(s * RPS, RPS)],
                        out0_hbm.at[pl.ds(s * RPS, RPS)])

    @pl.when(c == 1)
    def _():
        pltpu.sync_copy(acc_sh.at[pl.ds(s * RPS, RPS)],
                        out1_hbm.at[pl.ds(s * RPS, RPS)])


BR = 1000
GR = N_NODES // BR


def _mm0_body(x_ref, w_ref, o_ref):
    o_ref[...] = jnp.dot(x_ref[...], w_ref[...],
                         preferred_element_type=jnp.float32)


_mm0 = pl.pallas_call(
    _mm0_body,
    grid=(GR,),
    in_specs=[pl.BlockSpec((BR, F), lambda i: (i, 0)),
              pl.BlockSpec((F, F), lambda i: (0, 0))],
    out_specs=pl.BlockSpec((BR, F), lambda i: (i, 0)),
    out_shape=jax.ShapeDtypeStruct((N_NODES, F), jnp.float32),
)


def _scale_body(h_ref, d0_ref, d1_ref, g_ref, dis_ref):
    deg = d0_ref[...] + d1_ref[...] + 1.0
    dis = lax.rsqrt(deg)
    dis_ref[...] = dis
    g_ref[...] = h_ref[...] * dis


_scale = pl.pallas_call(
    _scale_body,
    grid=(GR,),
    in_specs=[pl.BlockSpec((BR, F), lambda i: (i, 0)),
              pl.BlockSpec((BR, 1), lambda i: (i, 0)),
              pl.BlockSpec((BR, 1), lambda i: (i, 0))],
    out_specs=[pl.BlockSpec((BR, F), lambda i: (i, 0)),
               pl.BlockSpec((BR, 1), lambda i: (i, 0))],
    out_shape=[jax.ShapeDtypeStruct((N_NODES, F), jnp.float32),
               jax.ShapeDtypeStruct((N_NODES, 1), jnp.float32)],
)


def _fuse_body(p0_ref, p1_ref, g0_ref, dis_ref, b_ref, o_ref):
    dis = dis_ref[...]
    h1 = dis * (p0_ref[...] + p1_ref[...] + g0_ref[...]) + b_ref[...]
    o_ref[...] = dis * jnp.maximum(h1, 0.0)


_fuse = pl.pallas_call(
    _fuse_body,
    grid=(GR,),
    in_specs=[pl.BlockSpec((BR, F), lambda i: (i, 0)),
              pl.BlockSpec((BR, F), lambda i: (i, 0)),
              pl.BlockSpec((BR, F), lambda i: (i, 0)),
              pl.BlockSpec((BR, 1), lambda i: (i, 0)),
              pl.BlockSpec((1, F), lambda i: (0, 0))],
    out_specs=pl.BlockSpec((BR, F), lambda i: (i, 0)),
    out_shape=jax.ShapeDtypeStruct((N_NODES, F), jnp.float32),
)


def _mm2_body(p0_ref, p1_ref, g1_ref, dis_ref, w_ref, b_ref, mean_ref, log_ref):
    q = dis_ref[...] * (p0_ref[...] + p1_ref[...] + g1_ref[...])
    out = jnp.dot(q, w_ref[...], preferred_element_type=jnp.float32) + b_ref[...]
    mean_ref[...] = out[:, :OUT_CH]
    log_ref[...] = out[:, OUT_CH:]


_mm2 = pl.pallas_call(
    _mm2_body,
    grid=(GR,),
    in_specs=[pl.BlockSpec((BR, F), lambda i: (i, 0)),
              pl.BlockSpec((BR, F), lambda i: (i, 0)),
              pl.BlockSpec((BR, F), lambda i: (i, 0)),
              pl.BlockSpec((BR, 1), lambda i: (i, 0)),
              pl.BlockSpec((F, F), lambda i: (0, 0)),
              pl.BlockSpec((1, F), lambda i: (0, 0))],
    out_specs=[pl.BlockSpec((BR, OUT_CH), lambda i: (i, 0)),
               pl.BlockSpec((BR, OUT_CH), lambda i: (i, 0))],
    out_shape=[jax.ShapeDtypeStruct((N_NODES, OUT_CH), jnp.float32),
               jax.ShapeDtypeStruct((N_NODES, OUT_CH), jnp.float32)],
)


@jax.jit
def _run(x, edge_index, W0, b0, Wm, bm, Wl, bl):
    src = edge_index[0].astype(jnp.int32)
    dst = edge_index[1].astype(jnp.int32)

    d0, d1 = _deg_kernel(dst)
    h0 = _mm0(x, W0)
    g0, dis = _scale(h0, d0[:N_NODES, None], d1[:N_NODES, None])
    s0a, s0b = _scat_kernel(g0, src, dst)
    g1 = _fuse(s0a, s0b, g0, dis, b0.reshape(1, F))
    s1a, s1b = _scat_kernel(g1, src, dst)
    W_ml = jnp.concatenate([Wm, Wl], axis=1)
    b_ml = jnp.concatenate([bm, bl]).reshape(1, F)
    mean, logstd = _mm2(s1a, s1b, g1, dis, W_ml, b_ml)
    return (mean, logstd)


def kernel(x, edge_index, W0, b0, Wm, bm, Wl, bl):
    return _run(x, edge_index, W0, b0, Wm, bm, Wl, bl)


# speedup vs baseline: 29.2367x; 1.7307x over previous
import dataclasses
import functools

import jax
import jax.numpy as jnp
from jax import lax
from jax.experimental import pallas as pl
from jax.experimental.pallas import tpu as pltpu
from jax.experimental.pallas import tpu_sc as plsc

N_NODES = 10000
N_EDGES = 320000
F = 128
OUT_CH = 64

NC, NS = 2, 16
NW = NC * NS
EPW = N_EDGES // NW
CHUNK = 100
NCHUNK = EPW // CHUNK
NPAIR = NCHUNK // 2
N_PAD = 10240
RPS = N_PAD // NS

_mesh = plsc.VectorSubcoreMesh(core_axis_name="c", subcore_axis_name="s")

_sc_params = pltpu.CompilerParams()
if "needs_layout_passes" in pltpu.CompilerParams.__dataclass_fields__:
    _sc_params = dataclasses.replace(_sc_params, needs_layout_passes=False)


@functools.partial(
    pl.kernel,
    out_type=[jax.ShapeDtypeStruct((N_PAD,), jnp.float32),
              jax.ShapeDtypeStruct((N_PAD,), jnp.float32)],
    mesh=_mesh,
    scratch_types=[
        pltpu.VMEM((EPW,), jnp.int32),
        pltpu.VMEM((N_PAD,), jnp.float32),
        pltpu.VMEM((RPS,), jnp.float32),
        pltpu.VMEM((RPS,), jnp.float32),
        pltpu.VMEM_SHARED((NS, N_PAD), jnp.float32),
    ],
    compiler_params=_sc_params,
)
def _deg_kernel(dst_hbm, out0_hbm, out1_hbm, dst_v, hist_v, red_v, tmp_v,
                acc_sh):
    c = lax.axis_index("c")
    s = lax.axis_index("s")
    wid = s * NC + c

    pltpu.sync_copy(dst_hbm.at[pl.ds(wid * EPW, EPW)], dst_v)

    @pl.loop(0, N_PAD // 16)
    def _(i):
        hist_v[pl.ds(i * 16, 16)] = jnp.zeros((16,), jnp.float32)

    ones16 = jnp.ones((16,), jnp.float32)

    @pl.loop(0, EPW // 16)
    def _(i):
        idx = dst_v[pl.ds(i * 16, 16)]
        plsc.addupdate_scatter(hist_v, [idx], ones16)

    pltpu.sync_copy(hist_v, acc_sh.at[s])
    plsc.subcore_barrier()

    @pl.loop(0, RPS // 16)
    def _(j):
        red_v[pl.ds(j * 16, 16)] = jnp.zeros((16,), jnp.float32)

    for k in range(NS):
        pltpu.sync_copy(acc_sh.at[k, pl.ds(s * RPS, RPS)], tmp_v)

        @pl.loop(0, RPS // 16)
        def _(j):
            red_v[pl.ds(j * 16, 16)] += tmp_v[pl.ds(j * 16, 16)]

    @pl.when(c == 0)
    def _():
        pltpu.sync_copy(red_v, out0_hbm.at[pl.ds(s * RPS, RPS)])

    @pl.when(c == 1)
    def _():
        pltpu.sync_copy(red_v, out1_hbm.at[pl.ds(s * RPS, RPS)])


@functools.partial(
    pl.kernel,
    out_type=[jax.ShapeDtypeStruct((N_PAD, F), jnp.float32),
              jax.ShapeDtypeStruct((N_PAD, F), jnp.float32)],
    mesh=_mesh,
    scratch_types=[
        pltpu.VMEM((NCHUNK // 2, CHUNK), jnp.int32),
        pltpu.VMEM((NCHUNK // 2, CHUNK), jnp.int32),
        [pltpu.VMEM((CHUNK, F), jnp.float32) for _ in range(2)],
        pltpu.VMEM_SHARED((N_PAD, F), jnp.float32),
        [pltpu.SemaphoreType.DMA for _ in range(2)],
        [pltpu.SemaphoreType.DMA for _ in range(2)],
    ],
    compiler_params=_sc_params,
)
def _scat_kernel(g_hbm, src4_hbm, dst4_hbm, z_hbm, out0_hbm, out1_hbm,
                 si_v, di_v, bufs, acc_sh, gsems, ssems):
    c = lax.axis_index("c")
    s = lax.axis_index("s")
    wid = s * NC + c
    half = NCHUNK // 2

    pltpu.sync_copy(src4_hbm.at[wid, 0], si_v)
    pltpu.sync_copy(dst4_hbm.at[wid, 0], di_v)
    pltpu.sync_copy(z_hbm.at[pl.ds(s * RPS, RPS)],
                    acc_sh.at[pl.ds(s * RPS, RPS)])
    plsc.subcore_barrier()

    for b in range(2):
        pltpu.async_copy(g_hbm.at[si_v.at[b]], bufs[b], gsems[b])

    @pl.loop(0, NPAIR)
    def _(p):
        k0 = p * 2
        scatters = []
        for b in range(2):
            pltpu.make_async_copy(g_hbm.at[si_v.at[0]], bufs[b],
                                  gsems[b]).wait()
            scatters.append(
                pltpu.async_copy(bufs[b],
                                 acc_sh.at[di_v.at[lax.rem(k0 + b, half)]],
                                 ssems[b], add=True))
        for b in range(2):
            scatters[b].wait()

        @pl.when(p == NPAIR // 2 - 1)
        def _():
            pltpu.sync_copy(src4_hbm.at[wid, 1], si_v)
            pltpu.sync_copy(dst4_hbm.at[wid, 1], di_v)

        for b in range(2):
            @pl.when(p < NPAIR - 1)
            def _():
                pltpu.async_copy(
                    g_hbm.at[si_v.at[lax.rem(k0 + 2 + b, half)]], bufs[b],
                    gsems[b])

    plsc.subcore_barrier()

    @pl.when(c == 0)
    def _():
        pltpu.sync_copy(acc_sh.at[pl.ds(s * RPS, RPS)],
                        out0_hbm.at[pl.ds(s * RPS, RPS)])

    @pl.when(c == 1)
    def _():
        pltpu.sync_copy(acc_sh.at[pl.ds(s * RPS, RPS)],
                        out1_hbm.at[pl.ds(s * RPS, RPS)])


BR = 1000
GR = N_NODES // BR


def _mm0_body(x_ref, w_ref, o_ref):
    o_ref[...] = jnp.dot(x_ref[...], w_ref[...],
                         preferred_element_type=jnp.float32)


_mm0 = pl.pallas_call(
    _mm0_body,
    grid=(GR,),
    in_specs=[pl.BlockSpec((BR, F), lambda i: (i, 0)),
              pl.BlockSpec((F, F), lambda i: (0, 0))],
    out_specs=pl.BlockSpec((BR, F), lambda i: (i, 0)),
    out_shape=jax.ShapeDtypeStruct((N_NODES, F), jnp.float32),
)


def _scale_body(h_ref, d0_ref, d1_ref, g_ref, dis_ref):
    deg = d0_ref[...] + d1_ref[...] + 1.0
    dis = lax.rsqrt(deg)
    dis_ref[...] = dis
    g_ref[...] = h_ref[...] * dis


_scale = pl.pallas_call(
    _scale_body,
    grid=(GR,),
    in_specs=[pl.BlockSpec((BR, F), lambda i: (i, 0)),
              pl.BlockSpec((BR, 1), lambda i: (i, 0)),
              pl.BlockSpec((BR, 1), lambda i: (i, 0))],
    out_specs=[pl.BlockSpec((BR, F), lambda i: (i, 0)),
               pl.BlockSpec((BR, 1), lambda i: (i, 0))],
    out_shape=[jax.ShapeDtypeStruct((N_NODES, F), jnp.float32),
               jax.ShapeDtypeStruct((N_NODES, 1), jnp.float32)],
)


def _fuse_body(p0_ref, p1_ref, g0_ref, dis_ref, b_ref, o_ref):
    dis = dis_ref[...]
    h1 = dis * (p0_ref[...] + p1_ref[...] + g0_ref[...]) + b_ref[...]
    o_ref[...] = dis * jnp.maximum(h1, 0.0)


_fuse = pl.pallas_call(
    _fuse_body,
    grid=(GR,),
    in_specs=[pl.BlockSpec((BR, F), lambda i: (i, 0)),
              pl.BlockSpec((BR, F), lambda i: (i, 0)),
              pl.BlockSpec((BR, F), lambda i: (i, 0)),
              pl.BlockSpec((BR, 1), lambda i: (i, 0)),
              pl.BlockSpec((1, F), lambda i: (0, 0))],
    out_specs=pl.BlockSpec((BR, F), lambda i: (i, 0)),
    out_shape=jax.ShapeDtypeStruct((N_NODES, F), jnp.float32),
)


def _mm2_body(p0_ref, p1_ref, g1_ref, dis_ref, w_ref, b_ref, mean_ref,
              log_ref):
    q = dis_ref[...] * (p0_ref[...] + p1_ref[...] + g1_ref[...])
    out = jnp.dot(q, w_ref[...], preferred_element_type=jnp.float32) + b_ref[...]
    mean_ref[...] = out[:, :OUT_CH]
    log_ref[...] = out[:, OUT_CH:]


_mm2 = pl.pallas_call(
    _mm2_body,
    grid=(GR,),
    in_specs=[pl.BlockSpec((BR, F), lambda i: (i, 0)),
              pl.BlockSpec((BR, F), lambda i: (i, 0)),
              pl.BlockSpec((BR, F), lambda i: (i, 0)),
              pl.BlockSpec((BR, 1), lambda i: (i, 0)),
              pl.BlockSpec((F, F), lambda i: (0, 0)),
              pl.BlockSpec((1, F), lambda i: (0, 0))],
    out_specs=[pl.BlockSpec((BR, OUT_CH), lambda i: (i, 0)),
               pl.BlockSpec((BR, OUT_CH), lambda i: (i, 0))],
    out_shape=[jax.ShapeDtypeStruct((N_NODES, OUT_CH), jnp.float32),
               jax.ShapeDtypeStruct((N_NODES, OUT_CH), jnp.float32)],
)


@jax.jit
def _run(x, edge_index, W0, b0, Wm, bm, Wl, bl):
    src = edge_index[0].astype(jnp.int32)
    dst = edge_index[1].astype(jnp.int32)
    src4 = src.reshape(NW, 2, NCHUNK // 2, CHUNK)
    dst4 = dst.reshape(NW, 2, NCHUNK // 2, CHUNK)
    zeros = jnp.zeros((N_PAD, F), jnp.float32)

    d0, d1 = _deg_kernel(dst)
    h0 = _mm0(x, W0)
    g0, dis = _scale(h0, d0[:N_NODES, None], d1[:N_NODES, None])
    s0a, s0b = _scat_kernel(g0, src4, dst4, zeros)
    g1 = _fuse(s0a, s0b, g0, dis, b0.reshape(1, F))
    s1a, s1b = _scat_kernel(g1, src4, dst4, zeros)
    W_ml = jnp.concatenate([Wm, Wl], axis=1)
    b_ml = jnp.concatenate([bm, bl]).reshape(1, F)
    mean, logstd = _mm2(s1a, s1b, g1, dis, W_ml, b_ml)
    return (mean, logstd)


def kernel(x, edge_index, W0, b0, Wm, bm, Wl, bl):
    return _run(x, edge_index, W0, b0, Wm, bm, Wl, bl)
